# baseline (speedup 1.0000x reference)
import jax
import jax.numpy as jnp
from jax import lax
from jax.experimental import pallas as pl
from jax.experimental.pallas import tpu as pltpu
from jax.experimental.pallas import tpu_sc as plsc

B = 16384
E = 64
VOCAB = 1000
TOK = 50
G = TOK // 2

NSPLIT = 2
BC = B // NSPLIT
NIDXC = BC * TOK

NC = 2
NS = 16
NW = NC * NS
BPW = BC // NW
PER_W = NIDXC // NW
CH = 128
NCHUNK = PER_W // CH
NSTR = 2
SUP = CH * NSTR
NSUP = PER_W // SUP
HSUP = NSUP // 2
VPG = 2 * BPW // 16

BM = 1024


def _sc_gather_body(idx_hbm, table_hbm, out_hbm,
                    idxb_v, idx_v, rows0, rows1,
                    sem_g0, sem_g1, sem_w0, sem_w1):
    wid = lax.axis_index("s") * NC + lax.axis_index("c")
    pltpu.sync_copy(idx_hbm.at[wid], idxb_v)

    it = lax.iota(jnp.int32, 16)
    patt = (it // 2) * TOK + (it % 2)
    for g in range(G):
        def permute(v2, carry, g=g):
            src = patt + (v2 * 8 * TOK + 2 * g)
            vec = plsc.load_gather(idxb_v, [src])
            r = g * (2 * BPW) + v2 * 16
            idx_v[r // CH, pl.ds(r % CH, 16)] = vec
            return carry
        lax.fori_loop(0, VPG, permute, 0)

    rows = (rows0, rows1)
    sem_g = (sem_g0, sem_g1)
    sem_w = (sem_w0, sem_w1)

    def start_gathers(s, q):
        for p in range(NSTR):
            pltpu.make_async_copy(
                table_hbm.at[idx_v.at[s * NSTR + p]],
                rows[q].at[pl.ds(p * CH, CH)],
                sem_g[q],
            ).start()

    def wait_gathers(q):
        pltpu.make_async_copy(
            out_hbm.at[pl.ds(0, SUP)], rows[q], sem_g[q]
        ).wait()

    def out_base(s):
        return (s // 2) * (2 * BC) + wid * (2 * BPW) + (s % 2) * SUP

    def wb(s, q):
        return pltpu.make_async_copy(
            rows[q], out_hbm.at[pl.ds(out_base(s), SUP)], sem_w[q]
        )

    start_gathers(0, 0)

    def body(ss, carry):
        s0 = ss * 2
        s1 = s0 + 1

        @pl.when(ss > 0)
        def _():
            wb(s0 - 1, 1).wait()

        start_gathers(s1, 1)
        wait_gathers(0)
        wb(s0, 0).start()
        wb(s0, 0).wait()

        @pl.when(ss < HSUP - 1)
        def _():
            start_gathers(s0 + 2, 0)

        wait_gathers(1)
        wb(s1, 1).start()
        return carry

    lax.fori_loop(0, HSUP, body, 0)
    wb(NSUP - 1, 1).wait()


def _sc_gather(conc_c, table):
    mesh = plsc.VectorSubcoreMesh(core_axis_name="c", subcore_axis_name="s")
    return pl.kernel(
        _sc_gather_body,
        out_type=jax.ShapeDtypeStruct((NIDXC, E), jnp.float32),
        mesh=mesh,
        compiler_params=pltpu.CompilerParams(
            use_tc_tiling_on_sc=False, needs_layout_passes=False
        ),
        scratch_types=[
            pltpu.VMEM((PER_W,), jnp.int32),
            pltpu.VMEM((NCHUNK, CH), jnp.int32),
            pltpu.VMEM((SUP, E), jnp.float32),
            pltpu.VMEM((SUP, E), jnp.float32),
            pltpu.SemaphoreType.DMA,
            pltpu.SemaphoreType.DMA,
            pltpu.SemaphoreType.DMA,
            pltpu.SemaphoreType.DMA,
        ],
    )(conc_c, table)


def _mm_body(x_ref, w_ref, b_ref, o_ref, x2_ref):
    for g in range(G):
        x2_ref[:, pl.ds(g * 2 * E, 2 * E)] = x_ref[g].astype(jnp.bfloat16)
    o_ref[...] = (
        jnp.dot(x2_ref[...], w_ref[...], preferred_element_type=jnp.float32)
        + b_ref[...]
    )


def _tc_matmul(x3, w, b2):
    return pl.pallas_call(
        _mm_body,
        grid=(BC // BM,),
        in_specs=[
            pl.BlockSpec((G, BM, 2 * E), lambda m: (0, m, 0)),
            pl.BlockSpec((TOK * E, VOCAB), lambda m: (0, 0)),
            pl.BlockSpec((1, VOCAB), lambda m: (0, 0)),
        ],
        out_specs=pl.BlockSpec((BM, VOCAB), lambda m: (m, 0)),
        out_shape=jax.ShapeDtypeStruct((BC, VOCAB), jnp.float32),
        scratch_shapes=[pltpu.VMEM((BM, TOK * E), jnp.bfloat16)],
    )(x3, w, b2)


def kernel(sentence, sentence_label, word_label, table, W, b):
    conc = jnp.concatenate([sentence, sentence_label], axis=1)
    concs = conc.reshape(NSPLIT, NW, PER_W)
    wb16 = W.astype(jnp.bfloat16)
    b2 = b.reshape(1, VOCAB)
    outs = []
    for c in range(NSPLIT):
        gathered = _sc_gather(concs[c], table)
        x3 = gathered.reshape(G, BC, 2 * E)
        outs.append(_tc_matmul(x3, wb16, b2))
    return jnp.concatenate(outs, axis=0)


# speedup vs baseline: 1.0948x; 1.0948x over previous
import functools

import jax
import jax.numpy as jnp
from jax import lax
from jax.experimental import pallas as pl
from jax.experimental.pallas import tpu as pltpu
from jax.experimental.pallas import tpu_sc as plsc

B = 16384
E = 64
VOCAB = 1000
L = 49
TOK = 50
G = TOK // 2

NSPLIT = 2
BC = B // NSPLIT
NIDXC = BC * TOK

NC = 2
NS = 16
NW = NC * NS
BPW = BC // NW
PER_W = NIDXC // NW
CH = 128
NCHUNK = PER_W // CH
NSTR = 2
SUP = CH * NSTR
NSUP = PER_W // SUP
HSUP = NSUP // 2
VPG = 2 * BPW // 16

BM = 1024


def _sc_gather_body(c, sent_hbm, lab_hbm, table_hbm, out_hbm,
                    sent_v, lab_v, idx_v, rows0, rows1,
                    sem_g0, sem_g1, sem_w0, sem_w1):
    wid = lax.axis_index("s") * NC + lax.axis_index("c")
    pltpu.sync_copy(sent_hbm.at[c].at[wid], sent_v)
    pltpu.sync_copy(lab_hbm.at[c].at[wid], lab_v)

    it = lax.iota(jnp.int32, 16)
    patt = (it // 2) * L + (it % 2)
    for g in range(G):
        def permute(v2, carry, g=g):
            if g < G - 1:
                vec = plsc.load_gather(sent_v, [patt + (v2 * 8 * L + 2 * g)])
            else:
                va = plsc.load_gather(
                    sent_v, [(it // 2) * L + (L - 1) + v2 * 8 * L])
                vb = plsc.load_gather(lab_v, [it // 2 + v2 * 8])
                vec = jnp.where((it % 2) == 0, va, vb)
            r = g * (2 * BPW) + v2 * 16
            idx_v[r // CH, pl.ds(r % CH, 16)] = vec
            return carry
        lax.fori_loop(0, VPG, permute, 0)

    rows = (rows0, rows1)
    sem_g = (sem_g0, sem_g1)
    sem_w = (sem_w0, sem_w1)

    def start_gathers(s, q):
        for p in range(NSTR):
            pltpu.make_async_copy(
                table_hbm.at[idx_v.at[s * NSTR + p]],
                rows[q].at[pl.ds(p * CH, CH)],
                sem_g[q],
            ).start()

    def wait_gathers(q):
        pltpu.make_async_copy(
            out_hbm.at[pl.ds(0, SUP)], rows[q], sem_g[q]
        ).wait()

    def out_base(s):
        return (s // 2) * (2 * BC) + wid * (2 * BPW) + (s % 2) * SUP

    def wb(s, q):
        return pltpu.make_async_copy(
            rows[q], out_hbm.at[pl.ds(out_base(s), SUP)], sem_w[q]
        )

    start_gathers(0, 0)

    def body(ss, carry):
        s0 = ss * 2
        s1 = s0 + 1

        @pl.when(ss > 0)
        def _():
            wb(s0 - 1, 1).wait()

        start_gathers(s1, 1)
        wait_gathers(0)
        wb(s0, 0).start()
        wb(s0, 0).wait()

        @pl.when(ss < HSUP - 1)
        def _():
            start_gathers(s0 + 2, 0)

        wait_gathers(1)
        wb(s1, 1).start()
        return carry

    lax.fori_loop(0, HSUP, body, 0)
    wb(NSUP - 1, 1).wait()


def _sc_gather(c, sent2, lab2, table):
    mesh = plsc.VectorSubcoreMesh(core_axis_name="c", subcore_axis_name="s")
    return pl.kernel(
        functools.partial(_sc_gather_body, c),
        out_type=jax.ShapeDtypeStruct((NIDXC, E), jnp.float32),
        mesh=mesh,
        compiler_params=pltpu.CompilerParams(
            use_tc_tiling_on_sc=False, needs_layout_passes=False
        ),
        scratch_types=[
            pltpu.VMEM((BPW * L,), jnp.int32),
            pltpu.VMEM((BPW,), jnp.int32),
            pltpu.VMEM((NCHUNK, CH), jnp.int32),
            pltpu.VMEM((SUP, E), jnp.float32),
            pltpu.VMEM((SUP, E), jnp.float32),
            pltpu.SemaphoreType.DMA,
            pltpu.SemaphoreType.DMA,
            pltpu.SemaphoreType.DMA,
            pltpu.SemaphoreType.DMA,
        ],
    )(sent2, lab2, table)


def _mm_body(x_ref, w_ref, b_ref, o_ref, x2_ref):
    for g in range(G):
        x2_ref[:, pl.ds(g * 2 * E, 2 * E)] = x_ref[g].astype(jnp.bfloat16)
    o_ref[...] = (
        jnp.dot(x2_ref[...], w_ref[...], preferred_element_type=jnp.float32)
        + b_ref[...]
    )


def _mm_body_acc(x_ref, w_ref, b_ref, prev_ref, o_ref, x2_ref):
    del prev_ref
    _mm_body(x_ref, w_ref, b_ref, o_ref, x2_ref)


def _tc_matmul(c, x3, w, b2, prev=None):
    moff = c * (BC // BM)
    in_specs = [
        pl.BlockSpec((G, BM, 2 * E), lambda m: (0, m, 0)),
        pl.BlockSpec((TOK * E, VOCAB), lambda m: (0, 0)),
        pl.BlockSpec((1, VOCAB), lambda m: (0, 0)),
    ]
    args = [x3, w, b2]
    body = _mm_body
    aliases = {}
    if prev is not None:
        in_specs.append(pl.BlockSpec(memory_space=pl.ANY))
        args.append(prev)
        body = _mm_body_acc
        aliases = {3: 0}
    return pl.pallas_call(
        body,
        grid=(BC // BM,),
        in_specs=in_specs,
        out_specs=pl.BlockSpec((BM, VOCAB), lambda m: (m + moff, 0)),
        out_shape=jax.ShapeDtypeStruct((B, VOCAB), jnp.float32),
        scratch_shapes=[pltpu.VMEM((BM, TOK * E), jnp.bfloat16)],
        input_output_aliases=aliases,
    )(*args)


def kernel(sentence, sentence_label, word_label, table, W, b):
    sent2 = sentence.reshape(NSPLIT, NW, BPW * L)
    lab2 = sentence_label.reshape(NSPLIT, NW, BPW)
    wb16 = W.astype(jnp.bfloat16)
    b2 = b.reshape(1, VOCAB)
    out = None
    for c in range(NSPLIT):
        gathered = _sc_gather(c, sent2, lab2, table)
        x3 = gathered.reshape(G, BC, 2 * E)
        out = _tc_matmul(c, x3, wb16, b2, out)
    return out
